# baseline (speedup 1.0000x reference)
import functools

import jax
import jax.numpy as jnp
from jax import lax
from jax.experimental import pallas as pl
from jax.experimental.pallas import tpu as pltpu
from jax.experimental.pallas import tpu_sc as plsc

_NUM_SC = 2
_NUM_TEC = 16
_NUM_WORKERS = _NUM_SC * _NUM_TEC

_CHUNK = 128
_K = 4


def _sc_gather(table, idx2d):
    n_chunks_all, chunk = idx2d.shape
    n_rows = n_chunks_all * chunk
    width = table.shape[1]
    per_w = n_rows // _NUM_WORKERS
    n_chunks = per_w // chunk
    n_super = n_chunks // _K

    mesh = plsc.VectorSubcoreMesh(core_axis_name="c", subcore_axis_name="s")

    @functools.partial(
        pl.kernel,
        mesh=mesh,
        out_type=jax.ShapeDtypeStruct((n_rows, width), jnp.float32),
        scratch_types=[
            pltpu.VMEM((n_chunks, chunk), jnp.int32),
            pltpu.VMEM((_K * chunk, width), jnp.float32),
            pltpu.VMEM((_K * chunk, width), jnp.float32),
            pltpu.SemaphoreType.DMA,
            pltpu.SemaphoreType.DMA,
            pltpu.SemaphoreType.DMA,
        ],
        compiler_params=pltpu.CompilerParams(use_tc_tiling_on_sc=False),
    )
    def gather_kernel(table_hbm, idx_hbm, out_hbm, idx_v, buf_a, buf_b,
                      sem_g, sem_sa, sem_sb):
        wid = lax.axis_index("s") * _NUM_SC + lax.axis_index("c")
        base_chunk = wid * n_chunks
        base_row = wid * per_w

        pltpu.sync_copy(idx_hbm.at[pl.ds(base_chunk, n_chunks)], idx_v)

        def fire_gathers(sstep, buf):
            for j in range(_K):
                pltpu.async_copy(
                    table_hbm.at[idx_v.at[sstep * _K + j]],
                    buf.at[pl.ds(j * chunk, chunk)], sem_g)

        def wait_gathers(buf):
            for j in range(_K):
                pltpu.make_async_copy(
                    table_hbm.at[idx_v.at[j]],
                    buf.at[pl.ds(j * chunk, chunk)], sem_g).wait()

        def store(sstep, buf, sem):
            return pltpu.async_copy(
                buf, out_hbm.at[pl.ds(base_row + sstep * _K * chunk,
                                      _K * chunk)], sem)

        def wait_store(buf, sem):
            pltpu.make_async_copy(
                buf, out_hbm.at[pl.ds(base_row, _K * chunk)], sem).wait()

        def body(q, carry):
            @pl.when(q > 0)
            def _():
                wait_store(buf_a, sem_sa)
            fire_gathers(2 * q, buf_a)
            wait_gathers(buf_a)
            store(2 * q, buf_a, sem_sa)
            @pl.when(q > 0)
            def _():
                wait_store(buf_b, sem_sb)
            fire_gathers(2 * q + 1, buf_b)
            wait_gathers(buf_b)
            store(2 * q + 1, buf_b, sem_sb)
            return carry

        lax.fori_loop(0, n_super // 2, body, 0)
        wait_store(buf_a, sem_sa)
        wait_store(buf_b, sem_sb)

    return gather_kernel(table, idx2d)


_TT = 2


def _bigru(embT, wrz_f, wn_f, urz_f, un_f, brz_f, bn_f, bhn_f,
           wrz_b, wn_b, urz_b, un_b, brz_b, bn_b, bhn_b, interpret=False):
    t_len, b, e = embT.shape
    h = un_f.shape[0]
    nt = t_len // _TT

    def body(xf_ref, xb_ref,
             wrzf_ref, wnf_ref, urzf_ref, unf_ref, brzf_ref, bnf_ref, bhnf_ref,
             wrzb_ref, wnb_ref, urzb_ref, unb_ref, brzb_ref, bnb_ref, bhnb_ref,
             outp_ref, hf_ref, hb_ref):
        @pl.when(pl.program_id(0) == 0)
        def _():
            hf_ref[...] = jnp.zeros_like(hf_ref)
            hb_ref[...] = jnp.zeros_like(hb_ref)

        def dot(a, w):
            return jnp.dot(a, w, preferred_element_type=jnp.float32)

        def gru_step(x_t, h_prev, wrz, wn, urz, un, brz, bn, bhn):
            rz = jax.nn.sigmoid(dot(x_t, wrz) + dot(h_prev, urz) + brz)
            r = rz[:, :h]
            z = rz[:, h:]
            hn = dot(h_prev, un) + bhn
            n = jnp.tanh(dot(x_t, wn) + bn + r * hn)
            return n + z * (h_prev - n)

        for i in range(_TT):
            h_f = gru_step(xf_ref[i], hf_ref[...], wrzf_ref[...], wnf_ref[...],
                           urzf_ref[...], unf_ref[...], brzf_ref[...],
                           bnf_ref[...], bhnf_ref[...])
            hf_ref[...] = h_f

            h_b = gru_step(xb_ref[_TT - 1 - i], hb_ref[...], wrzb_ref[...],
                           wnb_ref[...], urzb_ref[...], unb_ref[...],
                           brzb_ref[...], bnb_ref[...], bhnb_ref[...])
            hb_ref[...] = h_b

            outp_ref[i] = jnp.concatenate([h_f, h_b], axis=1)

    full = lambda shape: pl.BlockSpec(shape, lambda j: (0,) * len(shape))
    wspecs = [full((e, 2 * h)), full((e, h)), full((h, 2 * h)), full((h, h)),
              full((1, 2 * h)), full((1, h)), full((1, h))]
    packed = pl.pallas_call(
        body,
        grid=(nt,),
        in_specs=[
            pl.BlockSpec((_TT, b, e), lambda j: (j, 0, 0)),
            pl.BlockSpec((_TT, b, e), lambda j: (nt - 1 - j, 0, 0)),
        ] + wspecs + wspecs,
        out_specs=pl.BlockSpec((_TT, b, 2 * h), lambda j: (j, 0, 0)),
        out_shape=jax.ShapeDtypeStruct((t_len, b, 2 * h), jnp.float32),
        scratch_shapes=[
            pltpu.VMEM((b, h), jnp.float32),
            pltpu.VMEM((b, h), jnp.float32),
        ],
        interpret=interpret,
    )(embT, embT,
      wrz_f, wn_f, urz_f, un_f, brz_f, bn_f, bhn_f,
      wrz_b, wn_b, urz_b, un_b, brz_b, bn_b, bhn_b)
    return packed


def _prep_weights(W_ih, W_hh, b_ih, b_hh, h):
    wi = jnp.transpose(W_ih)
    wh = jnp.transpose(W_hh)
    wrz = wi[:, :2 * h]
    wn = wi[:, 2 * h:]
    urz = wh[:, :2 * h]
    un = wh[:, 2 * h:]
    brz = (b_ih[:2 * h] + b_hh[:2 * h]).reshape(1, 2 * h)
    bn = b_ih[2 * h:].reshape(1, h)
    bhn = b_hh[2 * h:].reshape(1, h)
    return wrz, wn, urz, un, brz, bn, bhn


def kernel(x, table, W_ih_f, W_hh_f, b_ih_f, b_hh_f,
           W_ih_b, W_hh_b, b_ih_b, b_hh_b):
    b, t_len = x.shape
    e = table.shape[1]
    h = W_hh_f.shape[1]

    x = x.astype(jnp.int32)
    idx2d = jnp.transpose(x).reshape(-1, _CHUNK)
    embT = _sc_gather(table, idx2d).reshape(t_len, b, e)

    packed = _bigru(
        embT,
        *_prep_weights(W_ih_f, W_hh_f, b_ih_f, b_hh_f, h),
        *_prep_weights(W_ih_b, W_hh_b, b_ih_b, b_hh_b, h),
    )
    out = jnp.concatenate([packed[:, :, :h], packed[::-1, :, h:]], axis=-1)
    return jnp.transpose(out, (1, 0, 2))


# speedup vs baseline: 1.3588x; 1.3588x over previous
import functools

import jax
import jax.numpy as jnp
from jax import lax
from jax.experimental import pallas as pl
from jax.experimental.pallas import tpu as pltpu
from jax.experimental.pallas import tpu_sc as plsc

_NUM_SC = 2
_NUM_TEC = 16
_NUM_WORKERS = _NUM_SC * _NUM_TEC

_CHUNK = 128
_K = 4


def _sc_gather(table, idx2d):
    n_chunks_all, chunk = idx2d.shape
    n_rows = n_chunks_all * chunk
    width = table.shape[1]
    per_w = n_rows // _NUM_WORKERS
    n_chunks = per_w // chunk
    n_super = n_chunks // _K

    mesh = plsc.VectorSubcoreMesh(core_axis_name="c", subcore_axis_name="s")

    @functools.partial(
        pl.kernel,
        mesh=mesh,
        out_type=jax.ShapeDtypeStruct((n_rows, width), jnp.float32),
        scratch_types=[
            pltpu.VMEM((n_chunks, chunk), jnp.int32),
            pltpu.VMEM((_K * chunk, width), jnp.float32),
            pltpu.VMEM((_K * chunk, width), jnp.float32),
            pltpu.SemaphoreType.DMA,
            pltpu.SemaphoreType.DMA,
            pltpu.SemaphoreType.DMA,
        ],
        compiler_params=pltpu.CompilerParams(use_tc_tiling_on_sc=False),
    )
    def gather_kernel(table_hbm, idx_hbm, out_hbm, idx_v, buf_a, buf_b,
                      sem_g, sem_sa, sem_sb):
        wid = lax.axis_index("s") * _NUM_SC + lax.axis_index("c")
        base_chunk = wid * n_chunks
        base_row = wid * per_w

        pltpu.sync_copy(idx_hbm.at[pl.ds(base_chunk, n_chunks)], idx_v)

        def fire_gathers(sstep, buf):
            for j in range(_K):
                pltpu.async_copy(
                    table_hbm.at[idx_v.at[sstep * _K + j]],
                    buf.at[pl.ds(j * chunk, chunk)], sem_g)

        def wait_gathers(buf):
            for j in range(_K):
                pltpu.make_async_copy(
                    table_hbm.at[idx_v.at[j]],
                    buf.at[pl.ds(j * chunk, chunk)], sem_g).wait()

        def store(sstep, buf, sem):
            return pltpu.async_copy(
                buf, out_hbm.at[pl.ds(base_row + sstep * _K * chunk,
                                      _K * chunk)], sem)

        def wait_store(buf, sem):
            pltpu.make_async_copy(
                buf, out_hbm.at[pl.ds(base_row, _K * chunk)], sem).wait()

        def body(q, carry):
            @pl.when(q > 0)
            def _():
                wait_store(buf_a, sem_sa)
            fire_gathers(2 * q, buf_a)
            wait_gathers(buf_a)
            store(2 * q, buf_a, sem_sa)
            @pl.when(q > 0)
            def _():
                wait_store(buf_b, sem_sb)
            fire_gathers(2 * q + 1, buf_b)
            wait_gathers(buf_b)
            store(2 * q + 1, buf_b, sem_sb)
            return carry

        lax.fori_loop(0, n_super // 2, body, 0)
        wait_store(buf_a, sem_sa)
        wait_store(buf_b, sem_sb)

    return gather_kernel(table, idx2d)


_TT = 2


def _bigru(embT, wrz_f, wn_f, urz_f, un_f, brz_f, bn_f, bhn_f,
           wrz_b, wn_b, urz_b, un_b, brz_b, bn_b, bhn_b, interpret=False):
    t_len, b, e = embT.shape
    h = un_f.shape[0]
    nt = t_len // _TT

    def body(xf_ref, xb_ref,
             wrzf_ref, wnf_ref, urzf_ref, unf_ref, brzf_ref, bnf_ref, bhnf_ref,
             wrzb_ref, wnb_ref, urzb_ref, unb_ref, brzb_ref, bnb_ref, bhnb_ref,
             outf_ref, outb_ref, hf_ref, hb_ref):
        @pl.when(pl.program_id(0) == 0)
        def _():
            hf_ref[...] = jnp.zeros_like(hf_ref)
            hb_ref[...] = jnp.zeros_like(hb_ref)

        def dot(a, w):
            return jnp.dot(a, w, preferred_element_type=jnp.float32)

        def gru_step(x_t, h_prev, wrz, wn, urz, un, brz, bn, bhn):
            rz = jax.nn.sigmoid(dot(x_t, wrz) + dot(h_prev, urz) + brz)
            r = rz[:, :h]
            z = rz[:, h:]
            hn = dot(h_prev, un) + bhn
            n = jnp.tanh(dot(x_t, wn) + bn + r * hn)
            return n + z * (h_prev - n)

        for i in range(_TT):
            h_f = gru_step(xf_ref[i], hf_ref[...], wrzf_ref[...], wnf_ref[...],
                           urzf_ref[...], unf_ref[...], brzf_ref[...],
                           bnf_ref[...], bhnf_ref[...])
            hf_ref[...] = h_f
            outf_ref[i] = h_f

            h_b = gru_step(xb_ref[_TT - 1 - i], hb_ref[...], wrzb_ref[...],
                           wnb_ref[...], urzb_ref[...], unb_ref[...],
                           brzb_ref[...], bnb_ref[...], bhnb_ref[...])
            hb_ref[...] = h_b
            outb_ref[_TT - 1 - i] = h_b

    full = lambda shape: pl.BlockSpec(shape, lambda j: (0,) * len(shape))
    wspecs = [full((e, 2 * h)), full((e, h)), full((h, 2 * h)), full((h, h)),
              full((1, 2 * h)), full((1, h)), full((1, h))]
    out_f, out_b = pl.pallas_call(
        body,
        grid=(nt,),
        in_specs=[
            pl.BlockSpec((_TT, b, e), lambda j: (j, 0, 0)),
            pl.BlockSpec((_TT, b, e), lambda j: (nt - 1 - j, 0, 0)),
        ] + wspecs + wspecs,
        out_specs=[
            pl.BlockSpec((_TT, b, h), lambda j: (j, 0, 0)),
            pl.BlockSpec((_TT, b, h), lambda j: (nt - 1 - j, 0, 0)),
        ],
        out_shape=[
            jax.ShapeDtypeStruct((t_len, b, h), jnp.float32),
            jax.ShapeDtypeStruct((t_len, b, h), jnp.float32),
        ],
        scratch_shapes=[
            pltpu.VMEM((b, h), jnp.float32),
            pltpu.VMEM((b, h), jnp.float32),
        ],
        interpret=interpret,
    )(embT, embT,
      wrz_f, wn_f, urz_f, un_f, brz_f, bn_f, bhn_f,
      wrz_b, wn_b, urz_b, un_b, brz_b, bn_b, bhn_b)
    return out_f, out_b


def _prep_weights(W_ih, W_hh, b_ih, b_hh, h):
    wi = jnp.transpose(W_ih)
    wh = jnp.transpose(W_hh)
    wrz = wi[:, :2 * h]
    wn = wi[:, 2 * h:]
    urz = wh[:, :2 * h]
    un = wh[:, 2 * h:]
    brz = (b_ih[:2 * h] + b_hh[:2 * h]).reshape(1, 2 * h)
    bn = b_ih[2 * h:].reshape(1, h)
    bhn = b_hh[2 * h:].reshape(1, h)
    return wrz, wn, urz, un, brz, bn, bhn


def kernel(x, table, W_ih_f, W_hh_f, b_ih_f, b_hh_f,
           W_ih_b, W_hh_b, b_ih_b, b_hh_b):
    b, t_len = x.shape
    e = table.shape[1]
    h = W_hh_f.shape[1]

    x = x.astype(jnp.int32)
    idx2d = jnp.transpose(x).reshape(-1, _CHUNK)
    embT = _sc_gather(table, idx2d).reshape(t_len, b, e)

    out_f, out_b = _bigru(
        embT,
        *_prep_weights(W_ih_f, W_hh_f, b_ih_f, b_hh_f, h),
        *_prep_weights(W_ih_b, W_hh_b, b_ih_b, b_hh_b, h),
    )
    out = jnp.concatenate([out_f, out_b], axis=-1)
    return jnp.transpose(out, (1, 0, 2))
